# Initial kernel scaffold; baseline (speedup 1.0000x reference)
import jax
import jax.numpy as jnp
from jax.experimental import pallas as pl


def kernel(review_x, review_to_rest_idx, Wq, bq, Wk, bk):
    raise NotImplementedError("write your pallas kernel here")


# speedup vs baseline: 12.5161x; 12.5161x over previous
import jax
import jax.numpy as jnp
from jax import lax
from jax.experimental import pallas as pl
from jax.experimental.pallas import tpu as pltpu
from jax.experimental.pallas import tpu_sc as plsc

NUM_SEG = 10000
TBL = 10240
LANES = 128
BLK = 3200


def _att_tc_body(x_ref, w_ref, b_ref, att_ref, gmax_ref):
    i = pl.program_id(0)
    qk = jnp.dot(x_ref[...], w_ref[...], preferred_element_type=jnp.float32)
    qk = qk + b_ref[...]
    h = qk.shape[1] // 2
    att = jnp.sum(qk[:, :h] * qk[:, h:], axis=1, keepdims=True)
    att_ref[...] = att

    @pl.when(i == 0)
    def _():
        gmax_ref[...] = jnp.full((8, 128), -jnp.inf, jnp.float32)

    gmax_ref[...] = jnp.maximum(gmax_ref[...], jnp.max(att))


def _att_tc(x, w, b):
    n = x.shape[0]
    d = x.shape[1]
    h2 = w.shape[1]
    grid = n // BLK
    return pl.pallas_call(
        _att_tc_body,
        grid=(grid,),
        in_specs=[
            pl.BlockSpec((BLK, d), lambda i: (i, 0)),
            pl.BlockSpec((d, h2), lambda i: (0, 0)),
            pl.BlockSpec((1, h2), lambda i: (0, 0)),
        ],
        out_specs=[
            pl.BlockSpec((BLK, 1), lambda i: (i, 0)),
            pl.BlockSpec((8, 128), lambda i: (0, 0)),
        ],
        out_shape=[
            jax.ShapeDtypeStruct((n, 1), jnp.float32),
            jax.ShapeDtypeStruct((8, 128), jnp.float32),
        ],
    )(x, w, b)


def _make_sc_softmax(rows):
    rows_per_tile = rows // 16
    out_rows = rows // 32
    stripe = TBL // 16

    def body(att_hbm, rid_hbm, gmax_hbm, out_hbm,
             e_v, rid_v, g_v, den_v, z_v, den_sh, sem):
        c = lax.axis_index("c")
        s = lax.axis_index("s")
        row0 = s * rows_per_tile
        pltpu.sync_copy(att_hbm.at[pl.ds(row0, rows_per_tile)], e_v)
        pltpu.sync_copy(rid_hbm.at[pl.ds(row0, rows_per_tile)], rid_v)
        pltpu.sync_copy(gmax_hbm, g_v)
        g = g_v[...]

        @pl.loop(0, stripe // 16)
        def _z(i):
            z_v[pl.ds(i * 16, 16)] = jnp.zeros((16,), jnp.float32)

        pltpu.sync_copy(z_v, den_sh.at[pl.ds(s * stripe, stripe)])

        @pl.loop(0, rows_per_tile)
        def _e(r):
            for v in range(8):
                sl = pl.ds(v * 16, 16)
                e_v[r, sl] = jnp.exp(e_v[r, sl] - g)

        plsc.subcore_barrier()

        @pl.loop(0, rows_per_tile // 8)
        def _sc(i):
            handles = []
            for k in range(8):
                r = i * 8 + k
                handles.append(
                    pltpu.async_copy(
                        e_v.at[r], den_sh.at[rid_v.at[r]], sem, add=True
                    )
                )
            for hd in handles:
                hd.wait()

        plsc.subcore_barrier()
        pltpu.sync_copy(den_sh, den_v)

        lr0 = out_rows * c

        @pl.loop(0, out_rows)
        def _o(r):
            lr = lr0 + r
            for v in range(8):
                sl = pl.ds(v * 16, 16)
                idx = rid_v[lr, sl]
                d = plsc.load_gather(den_v, [idx])
                e_v[lr, sl] = e_v[lr, sl] / d

        pltpu.sync_copy(
            e_v.at[pl.ds(lr0, out_rows)],
            out_hbm.at[pl.ds(row0 + lr0, out_rows)],
        )

    return pl.kernel(
        body,
        out_type=jax.ShapeDtypeStruct((rows, LANES), jnp.float32),
        mesh=plsc.VectorSubcoreMesh(core_axis_name="c", subcore_axis_name="s"),
        compiler_params=pltpu.CompilerParams(needs_layout_passes=False),
        scratch_types=[
            pltpu.VMEM((rows_per_tile, LANES), jnp.float32),
            pltpu.VMEM((rows_per_tile, LANES), jnp.int32),
            pltpu.VMEM((16,), jnp.float32),
            pltpu.VMEM((TBL,), jnp.float32),
            pltpu.VMEM((stripe,), jnp.float32),
            pltpu.VMEM_SHARED((TBL,), jnp.float32),
            pltpu.SemaphoreType.DMA,
        ],
    )


def kernel(review_x, review_to_rest_idx, Wq, bq, Wk, bk):
    n = review_x.shape[0]
    w = jnp.concatenate([Wq, Wk], axis=1)
    b = jnp.concatenate([bq, bk])[None, :]
    att, gmax8 = _att_tc(review_x, w, b)

    rows_real = n // LANES
    rows = ((rows_real + 255) // 256) * 256
    pad_rows = rows - rows_real
    att2d = att.reshape(rows_real, LANES)
    att2d = jnp.pad(att2d, ((0, pad_rows), (0, 0)), constant_values=-1e30)
    rid2d = jnp.pad(review_to_rest_idx, (0, pad_rows * LANES)).reshape(
        rows, LANES
    )
    gvec = gmax8[0, :16]

    w2d = _make_sc_softmax(rows)(att2d, rid2d, gvec)
    return w2d.reshape(-1)[:n]
